# Initial kernel scaffold; baseline (speedup 1.0000x reference)
import jax
import jax.numpy as jnp
from jax.experimental import pallas as pl


def kernel(x, edge_index, params):
    raise NotImplementedError("write your pallas kernel here")


# speedup vs baseline: 7.3671x; 7.3671x over previous
import functools

import jax
import jax.numpy as jnp
from jax import lax
from jax.experimental import pallas as pl
from jax.experimental.pallas import tpu as pltpu
from jax.experimental.pallas import tpu_sc as plsc

N = 10000
E = 320000
FEAT = 128
HID = 128
EMB = 128

NC = 2
NS = 16
NW = NC * NS
EPW = E // NW
CHUNK = 125
NCHUNK = EPW // CHUNK
ACC_N = 10240
RPT = ACC_N // NS

def _sc_agg_body(h_hbm, src_hbm, dst_hbm, zeros_hbm, out_hbm,
                 src_v, dst_v, rows_v, acc_sh, sem):
    c = lax.axis_index("c")
    s = lax.axis_index("s")
    pltpu.sync_copy(zeros_hbm, acc_sh.at[pl.ds(s * RPT, RPT)])
    base = (c * NS + s) * NCHUNK
    pltpu.sync_copy(src_hbm.at[pl.ds(base, NCHUNK)], src_v)
    pltpu.sync_copy(dst_hbm.at[pl.ds(base, NCHUNK)], dst_v)
    plsc.subcore_barrier()

    def body(j, carry):
        pltpu.async_copy(h_hbm.at[src_v.at[j]], rows_v, sem).wait()
        pltpu.sync_copy(rows_v, acc_sh.at[dst_v.at[j]], add=True)
        return carry

    lax.fori_loop(0, NCHUNK, body, 0)
    plsc.subcore_barrier()
    pltpu.sync_copy(acc_sh.at[pl.ds(s * RPT, RPT)],
                    out_hbm.at[c].at[pl.ds(s * RPT, RPT)])


@functools.cache
def _sc_agg_call():
    mesh = plsc.VectorSubcoreMesh(core_axis_name="c", subcore_axis_name="s",
                                  num_cores=NC, num_subcores=NS)
    return pl.kernel(
        _sc_agg_body,
        out_type=jax.ShapeDtypeStruct((NC, ACC_N, HID), jnp.float32),
        mesh=mesh,
        scratch_types=[
            pltpu.VMEM((NCHUNK, CHUNK), jnp.int32),
            pltpu.VMEM((NCHUNK, CHUNK), jnp.int32),
            pltpu.VMEM((CHUNK, HID), jnp.float32),
            pltpu.VMEM_SHARED((ACC_N, HID), jnp.float32),
            pltpu.SemaphoreType.DMA,
        ],
    )


def _conv_body(h_ref, agg_ref, w1_ref, b1_ref, gnw_ref, gnb_ref, gnms_ref,
               w2_ref, b2_ref, out_ref):
    y = h_ref[...] + agg_ref[0, :N] + agg_ref[1, :N]
    z = jnp.dot(y, w1_ref[...], preferred_element_type=jnp.float32) + b1_ref[...]
    mean = jnp.mean(z, axis=0, keepdims=True)
    ctr = z - mean * gnms_ref[...]
    var = jnp.mean(ctr * ctr, axis=0, keepdims=True)
    zn = gnw_ref[...] * ctr * lax.rsqrt(var + 1e-5) + gnb_ref[...]
    zr = jnp.maximum(zn, 0.0)
    h2 = jnp.dot(zr, w2_ref[...], preferred_element_type=jnp.float32) + b2_ref[...]
    out_ref[...] = jnp.maximum(h2, 0.0)


_conv_call = pl.pallas_call(
    _conv_body,
    out_shape=jax.ShapeDtypeStruct((N, HID), jnp.float32),
)


def _mlp_body(x_ref, h1_ref, h2_ref, h3_ref, w1_ref, b1_ref, bnw_ref, bnb_ref,
              w2_ref, b2_ref, out_ref):
    z = (jnp.dot(x_ref[...], w1_ref[0], preferred_element_type=jnp.float32)
         + jnp.dot(h1_ref[...], w1_ref[1], preferred_element_type=jnp.float32)
         + jnp.dot(h2_ref[...], w1_ref[2], preferred_element_type=jnp.float32)
         + jnp.dot(h3_ref[...], w1_ref[3], preferred_element_type=jnp.float32)
         + b1_ref[...])
    mean = jnp.mean(z, axis=0, keepdims=True)
    ctr = z - mean
    var = jnp.mean(ctr * ctr, axis=0, keepdims=True)
    zn = bnw_ref[...] * ctr * lax.rsqrt(var + 1e-5) + bnb_ref[...]
    zr = jnp.maximum(zn, 0.0)
    out_ref[...] = (jnp.dot(zr, w2_ref[...], preferred_element_type=jnp.float32)
                    + b2_ref[...])


_mlp_call = pl.pallas_call(
    _mlp_body,
    out_shape=jax.ShapeDtypeStruct((N, EMB), jnp.float32),
)


def kernel(x, edge_index, params):
    src2d = edge_index[0].reshape(E // CHUNK, CHUNK)
    dst2d = edge_index[1].reshape(E // CHUNK, CHUNK)
    zeros = jnp.zeros((RPT, HID), jnp.float32)

    h = x
    hs = [x]
    for i in range(3):
        p = params['conv%d' % i]
        agg = _sc_agg_call()(h, src2d, dst2d, zeros)
        h = _conv_call(h, agg, p['W1'], p['b1'].reshape(1, HID),
                       p['gn_w'].reshape(1, HID), p['gn_b'].reshape(1, HID),
                       p['gn_ms'].reshape(1, HID), p['W2'],
                       p['b2'].reshape(1, HID))
        hs.append(h)

    m = params['mlp']
    return _mlp_call(hs[0], hs[1], hs[2], hs[3],
                     m['W1'].reshape(4, HID, HID), m['b1'].reshape(1, HID),
                     m['bn_w'].reshape(1, HID), m['bn_b'].reshape(1, HID),
                     m['W2'], m['b2'].reshape(1, EMB))


# speedup vs baseline: 9.3704x; 1.2719x over previous
import functools

import jax
import jax.numpy as jnp
from jax import lax
from jax.experimental import pallas as pl
from jax.experimental.pallas import tpu as pltpu
from jax.experimental.pallas import tpu_sc as plsc

N = 10000
E = 320000
FEAT = 128
HID = 128
EMB = 128

NC = 2
NS = 16
NW = NC * NS
EPW = E // NW
CHUNK = 125
NCHUNK = EPW // CHUNK
ACC_N = 10240
RPT = ACC_N // NS

NCH_H = NCHUNK // 2
NITER = NCH_H // 2


def _sc_agg_body(h_hbm, src_hbm, dst_hbm, zeros_hbm, out_hbm,
                 src_v, dst_v, rows0, rows1, acc_sh, gsem, ssem):
    c = lax.axis_index("c")
    s = lax.axis_index("s")
    pltpu.sync_copy(zeros_hbm, acc_sh.at[pl.ds(s * RPT, RPT)])
    plsc.subcore_barrier()

    rows = (rows0, rows1)
    gsems = (gsem.at[0], gsem.at[1])
    ssems = (ssem.at[0], ssem.at[1])

    def start_g(j, b):
        pltpu.async_copy(h_hbm.at[src_v.at[j]], rows[b], gsems[b])

    def drain_g(j, b):
        pltpu.make_async_copy(h_hbm.at[src_v.at[j]], rows[b], gsems[b]).wait()

    def start_s(j, b):
        pltpu.async_copy(rows[b], acc_sh.at[dst_v.at[j]], ssems[b], add=True)

    def drain_s(j, b):
        pltpu.make_async_copy(rows[b], acc_sh.at[dst_v.at[j]],
                              ssems[b]).wait()

    for half in range(2):
        base = (c * NS + s) * NCHUNK + half * NCH_H
        pltpu.sync_copy(src_hbm.at[pl.ds(base, NCH_H)], src_v)
        pltpu.sync_copy(dst_hbm.at[pl.ds(base, NCH_H)], dst_v)

        start_g(0, 0)

        def body(t, carry):
            a = 2 * t
            drain_g(a, 0)
            start_s(a, 0)

            @pl.when(t > 0)
            def _():
                drain_s(a - 1, 1)

            start_g(a + 1, 1)
            drain_g(a + 1, 1)
            start_s(a + 1, 1)
            drain_s(a, 0)

            @pl.when(t < NITER - 1)
            def _():
                start_g(a + 2, 0)

            return carry

        lax.fori_loop(0, NITER, body, 0)
        drain_s(NCH_H - 1, 1)

    plsc.subcore_barrier()
    pltpu.sync_copy(acc_sh.at[pl.ds(s * RPT, RPT)],
                    out_hbm.at[c].at[pl.ds(s * RPT, RPT)])


@functools.cache
def _sc_agg_call():
    mesh = plsc.VectorSubcoreMesh(core_axis_name="c", subcore_axis_name="s",
                                  num_cores=NC, num_subcores=NS)
    return pl.kernel(
        _sc_agg_body,
        out_type=jax.ShapeDtypeStruct((NC, ACC_N, HID), jnp.float32),
        mesh=mesh,
        scratch_types=[
            pltpu.VMEM((NCH_H, CHUNK), jnp.int32),
            pltpu.VMEM((NCH_H, CHUNK), jnp.int32),
            pltpu.VMEM((CHUNK, HID), jnp.float32),
            pltpu.VMEM((CHUNK, HID), jnp.float32),
            pltpu.VMEM_SHARED((ACC_N, HID), jnp.float32),
            pltpu.SemaphoreType.DMA((2,)),
            pltpu.SemaphoreType.DMA((2,)),
        ],
    )


def _conv_body(h_ref, agg_ref, w1_ref, b1_ref, gnw_ref, gnb_ref, gnms_ref,
               w2_ref, b2_ref, out_ref):
    y = h_ref[...] + agg_ref[0, :N] + agg_ref[1, :N]
    z = jnp.dot(y, w1_ref[...], preferred_element_type=jnp.float32) + b1_ref[...]
    mean = jnp.mean(z, axis=0, keepdims=True)
    ctr = z - mean * gnms_ref[...]
    var = jnp.mean(ctr * ctr, axis=0, keepdims=True)
    zn = gnw_ref[...] * ctr * lax.rsqrt(var + 1e-5) + gnb_ref[...]
    zr = jnp.maximum(zn, 0.0)
    h2 = jnp.dot(zr, w2_ref[...], preferred_element_type=jnp.float32) + b2_ref[...]
    out_ref[...] = jnp.maximum(h2, 0.0)


_conv_call = pl.pallas_call(
    _conv_body,
    out_shape=jax.ShapeDtypeStruct((N, HID), jnp.float32),
)


def _mlp_body(x_ref, h1_ref, h2_ref, h3_ref, w1_ref, b1_ref, bnw_ref, bnb_ref,
              w2_ref, b2_ref, out_ref):
    z = (jnp.dot(x_ref[...], w1_ref[0], preferred_element_type=jnp.float32)
         + jnp.dot(h1_ref[...], w1_ref[1], preferred_element_type=jnp.float32)
         + jnp.dot(h2_ref[...], w1_ref[2], preferred_element_type=jnp.float32)
         + jnp.dot(h3_ref[...], w1_ref[3], preferred_element_type=jnp.float32)
         + b1_ref[...])
    mean = jnp.mean(z, axis=0, keepdims=True)
    ctr = z - mean
    var = jnp.mean(ctr * ctr, axis=0, keepdims=True)
    zn = bnw_ref[...] * ctr * lax.rsqrt(var + 1e-5) + bnb_ref[...]
    zr = jnp.maximum(zn, 0.0)
    out_ref[...] = (jnp.dot(zr, w2_ref[...], preferred_element_type=jnp.float32)
                    + b2_ref[...])


_mlp_call = pl.pallas_call(
    _mlp_body,
    out_shape=jax.ShapeDtypeStruct((N, EMB), jnp.float32),
)


def kernel(x, edge_index, params):
    src2d = edge_index[0].reshape(E // CHUNK, CHUNK)
    dst2d = edge_index[1].reshape(E // CHUNK, CHUNK)
    zeros = jnp.zeros((RPT, HID), jnp.float32)

    h = x
    hs = [x]
    for i in range(3):
        p = params['conv%d' % i]
        agg = _sc_agg_call()(h, src2d, dst2d, zeros)
        h = _conv_call(h, agg, p['W1'], p['b1'].reshape(1, HID),
                       p['gn_w'].reshape(1, HID), p['gn_b'].reshape(1, HID),
                       p['gn_ms'].reshape(1, HID), p['W2'],
                       p['b2'].reshape(1, HID))
        hs.append(h)

    m = params['mlp']
    return _mlp_call(hs[0], hs[1], hs[2], hs[3],
                     m['W1'].reshape(4, HID, HID), m['b1'].reshape(1, HID),
                     m['bn_w'].reshape(1, HID), m['bn_b'].reshape(1, HID),
                     m['W2'], m['b2'].reshape(1, EMB))
